# baseline (speedup 1.0000x reference)
import functools

import jax
import jax.numpy as jnp
from jax import lax
from jax.experimental import pallas as pl
from jax.experimental.pallas import tpu as pltpu
from jax.experimental.pallas import tpu_sc as plsc

N = 100000
D1 = 512
D2 = 768
M = 1000
K = 10
RB = 1024
GA = 98
NPAD = GA * RB
NW = 16
CHUNK = NPAD // NW
CPW = 1024 // NW
EQ_CAP = 1008
NEG_INF = float("-inf")
IMIN = -2**31


def _sortable_key(s):
    bits = lax.bitcast_convert_type(s, jnp.int32)
    m = lax.shift_right_arithmetic(bits, 31)
    return bits ^ (m & jnp.int32(0x7FFFFFFF))


def _stage1_body(x_ref, t1_ref, keys_out, meta_out, skeys):
    b = pl.program_id(0)
    x = x_ref[...]
    t1 = t1_ref[...]
    dot = jax.lax.dot_general(x, t1.reshape(D1, 1), (((1,), (0,)), ((), ())),
                              preferred_element_type=jnp.float32)[:, 0]
    x2 = x * x
    ones = jnp.full((D1, 1), 1.0, dtype=jnp.float32)
    na = jnp.sqrt(jax.lax.dot_general(x2, ones, (((1,), (0,)), ((), ())),
                                      preferred_element_type=jnp.float32)[:, 0])
    nb = jnp.sqrt(jnp.sum(t1 * t1))
    s = dot / jnp.maximum(na * nb, jnp.float32(1e-8))
    key = _sortable_key(s)
    rows = b * RB + lax.broadcasted_iota(jnp.int32, (RB,), 0)
    key = jnp.where(rows < N, key, jnp.int32(IMIN))
    keys_out[...] = key
    skeys[pl.ds(8 * b, 8), :] = key.reshape(8, 128)

    @pl.when(b == GA - 1)
    def _():
        allk = skeys[...]

        def cnt_ge(t):
            return jnp.sum((allk >= t).astype(jnp.int32))

        def bs_step(_, c):
            lo, hi = c
            mid = (lo >> 1) + (hi >> 1) + (lo & hi & 1)
            ge = cnt_ge(mid) >= M
            return jnp.where(ge, mid, lo), jnp.where(ge, hi, mid)

        lo, hi = lax.fori_loop(0, 32, bs_step, (jnp.int32(IMIN), jnp.int32(2**31 - 1)))
        t_key = lo
        c_gt = cnt_ge(t_key + 1)
        t_need = M - c_gt
        i = lax.broadcasted_iota(jnp.int32, (128,), 0)
        meta = jnp.where(i == 0, t_key, jnp.where(i == 1, c_gt, jnp.where(i == 2, t_need, 0)))
        meta_out[...] = meta


_stage1 = pl.pallas_call(
    _stage1_body,
    grid=(GA,),
    in_specs=[
        pl.BlockSpec((RB, D1), lambda b: (b, 0)),
        pl.BlockSpec((D1,), lambda b: (0,)),
    ],
    out_specs=[
        pl.BlockSpec((RB,), lambda b: (b,)),
        pl.BlockSpec((128,), lambda b: (0,)),
    ],
    out_shape=[
        jax.ShapeDtypeStruct((NPAD,), jnp.int32),
        jax.ShapeDtypeStruct((128,), jnp.int32),
    ],
    scratch_shapes=[pltpu.VMEM((GA * 8, 128), jnp.int32)],
)


def _lane(vec, i):
    return jnp.sum(jnp.where(lax.iota(jnp.int32, 16) == i, vec, 0))


def _sc_body(keys_hbm, meta_hbm, stage2_hbm, cand_hbm, emb_hbm,
             keys_v, gt_v, eq_v, idx_v, meta_v, cnts_v, ids_v, rows_v,
             shared_cnts, shared_cand, sem):
    wid = lax.axis_index("s")
    lane16 = lax.iota(jnp.int32, 16)

    pltpu.sync_copy(keys_hbm.at[pl.ds(wid * CHUNK, CHUNK)], keys_v)
    pltpu.sync_copy(meta_hbm.at[pl.ds(0, 16)], meta_v)
    mv = meta_v[...]
    t_key = _lane(mv, 0)
    c_gt_tot = _lane(mv, 1)
    t_need = _lane(mv, 2)
    t_vec = jnp.full((16,), t_key, dtype=jnp.int32)

    def comp_step(i, carry):
        cg_vec, ce_vec = carry
        v = keys_v[pl.ds(i * 16, 16)]
        gids = wid * CHUNK + i * 16 + lane16
        m_gt = v > t_vec
        m_eq = v == t_vec
        pos_g = cg_vec + plsc.cumsum(m_gt.astype(jnp.int32)) - 1
        plsc.store_scatter(gt_v, [pos_g], gids, mask=m_gt)
        pos_e = ce_vec + plsc.cumsum(m_eq.astype(jnp.int32)) - 1
        pos_e = jnp.minimum(pos_e, EQ_CAP + lane16)
        plsc.store_scatter(eq_v, [pos_e], gids, mask=m_eq)
        cg_vec = cg_vec + plsc.all_reduce_population_count(m_gt)
        ce_vec = ce_vec + plsc.all_reduce_population_count(m_eq)
        return cg_vec, ce_vec

    zero16 = jnp.zeros((16,), jnp.int32)
    cg_vec, ce_vec = lax.fori_loop(0, CHUNK // 16, comp_step, (zero16, zero16))
    ce_vec = jnp.minimum(ce_vec, EQ_CAP)

    cnts_v[pl.ds(0, 16)] = cg_vec
    cnts_v[pl.ds(16, 16)] = ce_vec
    pltpu.sync_copy(cnts_v.at[pl.ds(0, 16)], shared_cnts.at[pl.ds(wid * 16, 16)])
    pltpu.sync_copy(cnts_v.at[pl.ds(16, 16)], shared_cnts.at[pl.ds((NW + wid) * 16, 16)])
    plsc.subcore_barrier()

    pltpu.sync_copy(shared_cnts, cnts_v)

    def pfx_step(w, carry):
        og, oe = carry
        cg = jnp.max(cnts_v[pl.ds(w * 16, 16)])
        ce = jnp.max(cnts_v[pl.ds((NW + w) * 16, 16)])
        take = (w < wid).astype(jnp.int32)
        return og + take * cg, oe + take * ce

    og, oe = lax.fori_loop(0, NW, pfx_step, (jnp.int32(0), jnp.int32(0)))
    cg_w = jnp.max(cg_vec)
    ce_w = jnp.max(ce_vec)
    acc_e = jnp.clip(t_need - oe, 0, ce_w)

    def scatter_list(list_ref, base, count, tag):
        def fill(p, carry):
            j = p * 16 + lane16
            dump = 1024 + ((wid * (37 + tag) + j) & 1023)
            idx_v[pl.ds(p * 16, 16)] = jnp.clip(
                jnp.where(j < count, base + j, dump), 0, 2047)
            return carry
        lax.fori_loop(0, 1024 // 16, fill, jnp.int32(0))
        pltpu.async_copy(list_ref, shared_cand.at[idx_v], sem).wait()

    scatter_list(gt_v, og, cg_w, 0)
    scatter_list(eq_v, c_gt_tot + oe, acc_e, 7)
    plsc.subcore_barrier()

    pltpu.sync_copy(shared_cand.at[pl.ds(wid * CPW, CPW)], ids_v)
    for q in range(CPW // 16):
        ids_v[pl.ds(q * 16, 16)] = jnp.clip(ids_v[pl.ds(q * 16, 16)], 0, N - 1)
    pltpu.sync_copy(ids_v, cand_hbm.at[pl.ds(wid * CPW, CPW)])
    pltpu.async_copy(stage2_hbm.at[ids_v], rows_v, sem).wait()
    pltpu.sync_copy(rows_v, emb_hbm.at[pl.ds(wid * CPW, CPW)])


@functools.cache
def _sc_stage():
    return pl.kernel(
        _sc_body,
        out_type=(jax.ShapeDtypeStruct((1024,), jnp.int32),
                  jax.ShapeDtypeStruct((1024, D2), jnp.float32)),
        mesh=plsc.VectorSubcoreMesh(core_axis_name="c", subcore_axis_name="s",
                                    num_cores=1, num_subcores=16),
        compiler_params=pltpu.CompilerParams(needs_layout_passes=False),
        scratch_types=[
            pltpu.VMEM((CHUNK,), jnp.int32),
            pltpu.VMEM((1024,), jnp.int32),
            pltpu.VMEM((1024,), jnp.int32),
            pltpu.VMEM((1024,), jnp.int32),
            pltpu.VMEM((16,), jnp.int32),
            pltpu.VMEM((2 * NW * 16,), jnp.int32),
            pltpu.VMEM((CPW,), jnp.int32),
            pltpu.VMEM((CPW, D2), jnp.float32),
            pltpu.VMEM_SHARED((2 * NW * 16,), jnp.int32),
            pltpu.VMEM_SHARED((2048,), jnp.int32),
            pltpu.SemaphoreType.DMA,
        ],
    )


def _stage2_body(emb_ref, t2_ref, cand_ref, out_ref):
    x = emb_ref[...]
    t2 = t2_ref[...]
    dot = jnp.sum(x * t2[None, :], axis=1)
    na = jnp.sqrt(jnp.sum(x * x, axis=1))
    nb = jnp.sqrt(jnp.sum(t2 * t2))
    s = (dot / jnp.maximum(na * nb, jnp.float32(1e-8))).reshape(8, 128)
    flat = (lax.broadcasted_iota(jnp.int32, (8, 128), 0) * 128
            + lax.broadcasted_iota(jnp.int32, (8, 128), 1))
    s = jnp.where(flat < M, s, jnp.float32(NEG_INF))
    cand = cand_ref[...].reshape(8, 128)
    acc = jnp.zeros((8, 128), jnp.int32)
    for k in range(K):
        mval = jnp.max(s)
        p = jnp.min(jnp.where(s == mval, flat, jnp.int32(4096)))
        gid = jnp.sum(jnp.where(flat == p, cand, 0))
        acc = jnp.where(flat == k, gid, acc)
        s = jnp.where(flat == p, jnp.float32(NEG_INF), s)
    out_ref[...] = acc


_stage2 = pl.pallas_call(
    _stage2_body,
    in_specs=[
        pl.BlockSpec((1024, D2), lambda: (0, 0)),
        pl.BlockSpec((D2,), lambda: (0,)),
        pl.BlockSpec((1024,), lambda: (0,)),
    ],
    out_specs=pl.BlockSpec((8, 128), lambda: (0, 0)),
    out_shape=jax.ShapeDtypeStruct((8, 128), jnp.int32),
)


def kernel(base_images_emb, stage2_images_emb, text_emb_stage1, text_emb_stage2, topm, topk):
    keys, meta = _stage1(base_images_emb, text_emb_stage1)
    cand, cand_emb = _sc_stage()(keys, meta, stage2_images_emb)
    out = _stage2(cand_emb, text_emb_stage2, cand)
    return out[0, :K]


# speedup vs baseline: 1.0646x; 1.0646x over previous
import functools

import jax
import jax.numpy as jnp
from jax import lax
from jax.experimental import pallas as pl
from jax.experimental.pallas import tpu as pltpu
from jax.experimental.pallas import tpu_sc as plsc

N = 100000
D1 = 512
D2 = 768
M = 1000
K = 10
RB = 1024
GA = 98
NPAD = GA * RB
NW = 16
CHUNK = NPAD // NW
CPW = 1024 // NW
EQ_CAP = 1008
NEG_INF = float("-inf")
IMIN = -2**31


def _sortable_key(s):
    bits = lax.bitcast_convert_type(s, jnp.int32)
    m = lax.shift_right_arithmetic(bits, 31)
    return bits ^ (m & jnp.int32(0x7FFFFFFF))


def _stage1_body(x_ref, t1_ref, keys_out, meta_out, skeys):
    b = pl.program_id(0)
    x = x_ref[...]
    t1 = t1_ref[...]
    dot = jnp.sum(x * t1[None, :], axis=1)
    na = jnp.sqrt(jnp.sum(x * x, axis=1))
    nb = jnp.sqrt(jnp.sum(t1 * t1))
    s = dot / jnp.maximum(na * nb, jnp.float32(1e-8))
    key = _sortable_key(s)
    rows = b * RB + lax.broadcasted_iota(jnp.int32, (RB,), 0)
    key = jnp.where(rows < N, key, jnp.int32(IMIN))
    keys_out[...] = key
    skeys[pl.ds(8 * b, 8), :] = key.reshape(8, 128)

    @pl.when(b == GA - 1)
    def _():
        allk = skeys[...]

        def cnt_ge(t):
            return jnp.sum((allk >= t).astype(jnp.int32))

        def bs_step(_, c):
            lo, hi = c
            mid = (lo >> 1) + (hi >> 1) + (lo & hi & 1)
            ge = cnt_ge(mid) >= M
            return jnp.where(ge, mid, lo), jnp.where(ge, hi, mid)

        lo, hi = lax.fori_loop(0, 32, bs_step, (jnp.int32(IMIN), jnp.int32(2**31 - 1)))
        t_key = lo
        c_gt = cnt_ge(t_key + 1)
        t_need = M - c_gt
        i = lax.broadcasted_iota(jnp.int32, (128,), 0)
        meta = jnp.where(i == 0, t_key, jnp.where(i == 1, c_gt, jnp.where(i == 2, t_need, 0)))
        meta_out[...] = meta


_stage1 = pl.pallas_call(
    _stage1_body,
    grid=(GA,),
    in_specs=[
        pl.BlockSpec((RB, D1), lambda b: (b, 0)),
        pl.BlockSpec((D1,), lambda b: (0,)),
    ],
    out_specs=[
        pl.BlockSpec((RB,), lambda b: (b,)),
        pl.BlockSpec((128,), lambda b: (0,)),
    ],
    out_shape=[
        jax.ShapeDtypeStruct((NPAD,), jnp.int32),
        jax.ShapeDtypeStruct((128,), jnp.int32),
    ],
    scratch_shapes=[pltpu.VMEM((GA * 8, 128), jnp.int32)],
)


def _lane(vec, i):
    return jnp.sum(jnp.where(lax.iota(jnp.int32, 16) == i, vec, 0))


def _sc_body(keys_hbm, meta_hbm, stage2_hbm, cand_hbm, emb_hbm,
             keys_v, gt_v, eq_v, idx_v, meta_v, cnts_v, ids_v, rows_v,
             shared_cnts, shared_cand, sem):
    wid = lax.axis_index("s")
    lane16 = lax.iota(jnp.int32, 16)

    pltpu.sync_copy(keys_hbm.at[pl.ds(wid * CHUNK, CHUNK)], keys_v)
    pltpu.sync_copy(meta_hbm.at[pl.ds(0, 16)], meta_v)
    mv = meta_v[...]
    t_key = _lane(mv, 0)
    c_gt_tot = _lane(mv, 1)
    t_need = _lane(mv, 2)
    t_vec = jnp.full((16,), t_key, dtype=jnp.int32)

    def comp_step(i, carry):
        cg_vec, ce_vec = carry
        v = keys_v[pl.ds(i * 16, 16)]
        gids = wid * CHUNK + i * 16 + lane16
        m_gt = v > t_vec
        m_eq = v == t_vec
        pos_g = cg_vec + plsc.cumsum(m_gt.astype(jnp.int32)) - 1
        plsc.store_scatter(gt_v, [pos_g], gids, mask=m_gt)
        pos_e = ce_vec + plsc.cumsum(m_eq.astype(jnp.int32)) - 1
        pos_e = jnp.minimum(pos_e, EQ_CAP + lane16)
        plsc.store_scatter(eq_v, [pos_e], gids, mask=m_eq)
        cg_vec = cg_vec + plsc.all_reduce_population_count(m_gt)
        ce_vec = ce_vec + plsc.all_reduce_population_count(m_eq)
        return cg_vec, ce_vec

    zero16 = jnp.zeros((16,), jnp.int32)
    cg_vec, ce_vec = lax.fori_loop(0, CHUNK // 16, comp_step, (zero16, zero16))
    ce_vec = jnp.minimum(ce_vec, EQ_CAP)

    cnts_v[pl.ds(0, 16)] = cg_vec
    cnts_v[pl.ds(16, 16)] = ce_vec
    pltpu.sync_copy(cnts_v.at[pl.ds(0, 16)], shared_cnts.at[pl.ds(wid * 16, 16)])
    pltpu.sync_copy(cnts_v.at[pl.ds(16, 16)], shared_cnts.at[pl.ds((NW + wid) * 16, 16)])
    plsc.subcore_barrier()

    pltpu.sync_copy(shared_cnts, cnts_v)

    def pfx_step(w, carry):
        og, oe = carry
        cg = jnp.max(cnts_v[pl.ds(w * 16, 16)])
        ce = jnp.max(cnts_v[pl.ds((NW + w) * 16, 16)])
        take = (w < wid).astype(jnp.int32)
        return og + take * cg, oe + take * ce

    og, oe = lax.fori_loop(0, NW, pfx_step, (jnp.int32(0), jnp.int32(0)))
    cg_w = jnp.max(cg_vec)
    ce_w = jnp.max(ce_vec)
    acc_e = jnp.clip(t_need - oe, 0, ce_w)

    def scatter_list(list_ref, base, count, tag):
        def fill(p, carry):
            j = p * 16 + lane16
            dump = 1024 + ((wid * (37 + tag) + j) & 1023)
            idx_v[pl.ds(p * 16, 16)] = jnp.clip(
                jnp.where(j < count, base + j, dump), 0, 2047)
            return carry
        lax.fori_loop(0, 1024 // 16, fill, jnp.int32(0))
        pltpu.async_copy(list_ref, shared_cand.at[idx_v], sem).wait()

    scatter_list(gt_v, og, cg_w, 0)
    scatter_list(eq_v, c_gt_tot + oe, acc_e, 7)
    plsc.subcore_barrier()

    pltpu.sync_copy(shared_cand.at[pl.ds(wid * CPW, CPW)], ids_v)
    for q in range(CPW // 16):
        ids_v[pl.ds(q * 16, 16)] = jnp.clip(ids_v[pl.ds(q * 16, 16)], 0, N - 1)
    pltpu.sync_copy(ids_v, cand_hbm.at[pl.ds(wid * CPW, CPW)])
    pltpu.async_copy(stage2_hbm.at[ids_v], rows_v, sem).wait()
    pltpu.sync_copy(rows_v, emb_hbm.at[pl.ds(wid * CPW, CPW)])


@functools.cache
def _sc_stage():
    return pl.kernel(
        _sc_body,
        out_type=(jax.ShapeDtypeStruct((1024,), jnp.int32),
                  jax.ShapeDtypeStruct((1024, D2), jnp.float32)),
        mesh=plsc.VectorSubcoreMesh(core_axis_name="c", subcore_axis_name="s",
                                    num_cores=1, num_subcores=16),
        compiler_params=pltpu.CompilerParams(needs_layout_passes=False),
        scratch_types=[
            pltpu.VMEM((CHUNK,), jnp.int32),
            pltpu.VMEM((1024,), jnp.int32),
            pltpu.VMEM((1024,), jnp.int32),
            pltpu.VMEM((1024,), jnp.int32),
            pltpu.VMEM((16,), jnp.int32),
            pltpu.VMEM((2 * NW * 16,), jnp.int32),
            pltpu.VMEM((CPW,), jnp.int32),
            pltpu.VMEM((CPW, D2), jnp.float32),
            pltpu.VMEM_SHARED((2 * NW * 16,), jnp.int32),
            pltpu.VMEM_SHARED((2048,), jnp.int32),
            pltpu.SemaphoreType.DMA,
        ],
    )


def _stage2_body(emb_ref, t2_ref, cand_ref, out_ref):
    x = emb_ref[...]
    t2 = t2_ref[...]
    dot = jnp.sum(x * t2[None, :], axis=1)
    na = jnp.sqrt(jnp.sum(x * x, axis=1))
    nb = jnp.sqrt(jnp.sum(t2 * t2))
    s = (dot / jnp.maximum(na * nb, jnp.float32(1e-8))).reshape(8, 128)
    flat = (lax.broadcasted_iota(jnp.int32, (8, 128), 0) * 128
            + lax.broadcasted_iota(jnp.int32, (8, 128), 1))
    s = jnp.where(flat < M, s, jnp.float32(NEG_INF))
    cand = cand_ref[...].reshape(8, 128)
    acc = jnp.zeros((8, 128), jnp.int32)
    for k in range(K):
        mval = jnp.max(s)
        p = jnp.min(jnp.where(s == mval, flat, jnp.int32(4096)))
        gid = jnp.sum(jnp.where(flat == p, cand, 0))
        acc = jnp.where(flat == k, gid, acc)
        s = jnp.where(flat == p, jnp.float32(NEG_INF), s)
    out_ref[...] = acc


_stage2 = pl.pallas_call(
    _stage2_body,
    in_specs=[
        pl.BlockSpec((1024, D2), lambda: (0, 0)),
        pl.BlockSpec((D2,), lambda: (0,)),
        pl.BlockSpec((1024,), lambda: (0,)),
    ],
    out_specs=pl.BlockSpec((8, 128), lambda: (0, 0)),
    out_shape=jax.ShapeDtypeStruct((8, 128), jnp.int32),
)


def kernel(base_images_emb, stage2_images_emb, text_emb_stage1, text_emb_stage2, topm, topk):
    keys, meta = _stage1(base_images_emb, text_emb_stage1)
    cand, cand_emb = _sc_stage()(keys, meta, stage2_images_emb)
    out = _stage2(cand_emb, text_emb_stage2, cand)
    return out[0, :K]


# speedup vs baseline: 1.2709x; 1.1938x over previous
import functools

import jax
import jax.numpy as jnp
from jax import lax
from jax.experimental import pallas as pl
from jax.experimental.pallas import tpu as pltpu
from jax.experimental.pallas import tpu_sc as plsc

N = 100000
D1 = 512
D2 = 768
M = 1000
K = 10
RB = 2048
GA = 49
NPAD = GA * RB
NW = 16
CHUNK = NPAD // NW
CPW = 1024 // NW
EQ_CAP = 1008
NEG_INF = float("-inf")
IMIN = -2**31


def _sortable_key(s):
    bits = lax.bitcast_convert_type(s, jnp.int32)
    m = lax.shift_right_arithmetic(bits, 31)
    return bits ^ (m & jnp.int32(0x7FFFFFFF))


def _stage1_body(x_ref, t1_ref, keys_out, meta_out, skeys):
    b = pl.program_id(0)
    x = x_ref[...]
    t1 = t1_ref[...]
    dot = jnp.sum(x * t1[None, :], axis=1)
    na = jnp.sqrt(jnp.sum(x * x, axis=1))
    nb = jnp.sqrt(jnp.sum(t1 * t1))
    s = dot / jnp.maximum(na * nb, jnp.float32(1e-8))
    key = _sortable_key(s)
    rows = b * RB + lax.broadcasted_iota(jnp.int32, (RB,), 0)
    key = jnp.where(rows < N, key, jnp.int32(IMIN))
    keys_out[...] = key
    skeys[pl.ds(16 * b, 16), :] = key.reshape(16, 128)

    @pl.when(b == GA - 1)
    def _():
        allk = skeys[...]

        def cnt_ge(t):
            return jnp.sum((allk >= t).astype(jnp.int32))

        def bs_step(_, c):
            lo, hi = c
            mid = (lo >> 1) + (hi >> 1) + (lo & hi & 1)
            ge = cnt_ge(mid) >= M
            return jnp.where(ge, mid, lo), jnp.where(ge, hi, mid)

        lo, hi = lax.fori_loop(0, 32, bs_step, (jnp.int32(IMIN), jnp.int32(2**31 - 1)))
        t_key = lo
        c_gt = cnt_ge(t_key + 1)
        t_need = M - c_gt
        i = lax.broadcasted_iota(jnp.int32, (128,), 0)
        meta = jnp.where(i == 0, t_key, jnp.where(i == 1, c_gt, jnp.where(i == 2, t_need, 0)))
        meta_out[...] = meta


_stage1 = pl.pallas_call(
    _stage1_body,
    grid=(GA,),
    in_specs=[
        pl.BlockSpec((RB, D1), lambda b: (b, 0)),
        pl.BlockSpec((D1,), lambda b: (0,)),
    ],
    out_specs=[
        pl.BlockSpec((RB,), lambda b: (b,)),
        pl.BlockSpec((128,), lambda b: (0,)),
    ],
    out_shape=[
        jax.ShapeDtypeStruct((NPAD,), jnp.int32),
        jax.ShapeDtypeStruct((128,), jnp.int32),
    ],
    scratch_shapes=[pltpu.VMEM((GA * 16, 128), jnp.int32)],
)


def _lane(vec, i):
    return jnp.sum(jnp.where(lax.iota(jnp.int32, 16) == i, vec, 0))


def _sc_body(keys_hbm, meta_hbm, stage2_hbm, cand_hbm, emb_hbm,
             keys_v, gt_v, eq_v, idx_v, meta_v, cnts_v, ids_v, rows_v,
             shared_cnts, shared_cand, sem):
    wid = lax.axis_index("s")
    lane16 = lax.iota(jnp.int32, 16)

    pltpu.sync_copy(keys_hbm.at[pl.ds(wid * CHUNK, CHUNK)], keys_v)
    pltpu.sync_copy(meta_hbm.at[pl.ds(0, 16)], meta_v)
    mv = meta_v[...]
    t_key = _lane(mv, 0)
    c_gt_tot = _lane(mv, 1)
    t_need = _lane(mv, 2)
    t_vec = jnp.full((16,), t_key, dtype=jnp.int32)

    def comp_step(i, carry):
        cg_vec, ce_vec = carry
        v = keys_v[pl.ds(i * 16, 16)]
        gids = wid * CHUNK + i * 16 + lane16
        m_gt = v > t_vec
        m_eq = v == t_vec
        pos_g = cg_vec + plsc.cumsum(m_gt.astype(jnp.int32)) - 1
        plsc.store_scatter(gt_v, [pos_g], gids, mask=m_gt)
        pos_e = ce_vec + plsc.cumsum(m_eq.astype(jnp.int32)) - 1
        pos_e = jnp.minimum(pos_e, EQ_CAP + lane16)
        plsc.store_scatter(eq_v, [pos_e], gids, mask=m_eq)
        cg_vec = cg_vec + plsc.all_reduce_population_count(m_gt)
        ce_vec = ce_vec + plsc.all_reduce_population_count(m_eq)
        return cg_vec, ce_vec

    zero16 = jnp.zeros((16,), jnp.int32)
    cg_vec, ce_vec = lax.fori_loop(0, CHUNK // 16, comp_step, (zero16, zero16))
    ce_vec = jnp.minimum(ce_vec, EQ_CAP)

    cnts_v[pl.ds(0, 16)] = cg_vec
    cnts_v[pl.ds(16, 16)] = ce_vec
    pltpu.sync_copy(cnts_v.at[pl.ds(0, 16)], shared_cnts.at[pl.ds(wid * 16, 16)])
    pltpu.sync_copy(cnts_v.at[pl.ds(16, 16)], shared_cnts.at[pl.ds((NW + wid) * 16, 16)])
    plsc.subcore_barrier()

    pltpu.sync_copy(shared_cnts, cnts_v)

    def pfx_step(w, carry):
        og, oe = carry
        cg = jnp.max(cnts_v[pl.ds(w * 16, 16)])
        ce = jnp.max(cnts_v[pl.ds((NW + w) * 16, 16)])
        take = (w < wid).astype(jnp.int32)
        return og + take * cg, oe + take * ce

    og, oe = lax.fori_loop(0, NW, pfx_step, (jnp.int32(0), jnp.int32(0)))
    cg_w = jnp.max(cg_vec)
    ce_w = jnp.max(ce_vec)
    acc_e = jnp.clip(t_need - oe, 0, ce_w)

    def scatter_list(list_ref, base, count, tag):
        def fill(p, carry):
            j = p * 16 + lane16
            dump = 1024 + ((wid * (37 + tag) + j) & 1023)
            idx_v[pl.ds(p * 16, 16)] = jnp.clip(
                jnp.where(j < count, base + j, dump), 0, 2047)
            return carry
        lax.fori_loop(0, 1024 // 16, fill, jnp.int32(0))
        pltpu.async_copy(list_ref, shared_cand.at[idx_v], sem).wait()

    scatter_list(gt_v, og, cg_w, 0)
    scatter_list(eq_v, c_gt_tot + oe, acc_e, 7)
    plsc.subcore_barrier()

    pltpu.sync_copy(shared_cand.at[pl.ds(wid * CPW, CPW)], ids_v)
    for q in range(CPW // 16):
        ids_v[pl.ds(q * 16, 16)] = jnp.clip(ids_v[pl.ds(q * 16, 16)], 0, N - 1)
    pltpu.sync_copy(ids_v, cand_hbm.at[pl.ds(wid * CPW, CPW)])
    pltpu.async_copy(stage2_hbm.at[ids_v], rows_v, sem).wait()
    pltpu.sync_copy(rows_v, emb_hbm.at[pl.ds(wid * CPW, CPW)])


@functools.cache
def _sc_stage():
    return pl.kernel(
        _sc_body,
        out_type=(jax.ShapeDtypeStruct((1024,), jnp.int32),
                  jax.ShapeDtypeStruct((1024, D2), jnp.float32)),
        mesh=plsc.VectorSubcoreMesh(core_axis_name="c", subcore_axis_name="s",
                                    num_cores=1, num_subcores=16),
        compiler_params=pltpu.CompilerParams(needs_layout_passes=False),
        scratch_types=[
            pltpu.VMEM((CHUNK,), jnp.int32),
            pltpu.VMEM((1024,), jnp.int32),
            pltpu.VMEM((1024,), jnp.int32),
            pltpu.VMEM((1024,), jnp.int32),
            pltpu.VMEM((16,), jnp.int32),
            pltpu.VMEM((2 * NW * 16,), jnp.int32),
            pltpu.VMEM((CPW,), jnp.int32),
            pltpu.VMEM((CPW, D2), jnp.float32),
            pltpu.VMEM_SHARED((2 * NW * 16,), jnp.int32),
            pltpu.VMEM_SHARED((2048,), jnp.int32),
            pltpu.SemaphoreType.DMA,
        ],
    )


def _stage2_body(emb_ref, t2_ref, cand_ref, out_ref):
    x = emb_ref[...]
    t2 = t2_ref[...]
    dot = jnp.sum(x * t2[None, :], axis=1)
    na = jnp.sqrt(jnp.sum(x * x, axis=1))
    nb = jnp.sqrt(jnp.sum(t2 * t2))
    s = (dot / jnp.maximum(na * nb, jnp.float32(1e-8))).reshape(8, 128)
    flat = (lax.broadcasted_iota(jnp.int32, (8, 128), 0) * 128
            + lax.broadcasted_iota(jnp.int32, (8, 128), 1))
    s = jnp.where(flat < M, s, jnp.float32(NEG_INF))
    cand = cand_ref[...].reshape(8, 128)
    acc = jnp.zeros((8, 128), jnp.int32)
    for k in range(K):
        mval = jnp.max(s)
        p = jnp.min(jnp.where(s == mval, flat, jnp.int32(4096)))
        gid = jnp.sum(jnp.where(flat == p, cand, 0))
        acc = jnp.where(flat == k, gid, acc)
        s = jnp.where(flat == p, jnp.float32(NEG_INF), s)
    out_ref[...] = acc


_stage2 = pl.pallas_call(
    _stage2_body,
    in_specs=[
        pl.BlockSpec((1024, D2), lambda: (0, 0)),
        pl.BlockSpec((D2,), lambda: (0,)),
        pl.BlockSpec((1024,), lambda: (0,)),
    ],
    out_specs=pl.BlockSpec((8, 128), lambda: (0, 0)),
    out_shape=jax.ShapeDtypeStruct((8, 128), jnp.int32),
)


def kernel(base_images_emb, stage2_images_emb, text_emb_stage1, text_emb_stage2, topm, topk):
    keys, meta = _stage1(base_images_emb, text_emb_stage1)
    cand, cand_emb = _sc_stage()(keys, meta, stage2_images_emb)
    out = _stage2(cand_emb, text_emb_stage2, cand)
    return out[0, :K]


# speedup vs baseline: 1.4530x; 1.1433x over previous
import functools

import jax
import jax.numpy as jnp
from jax import lax
from jax.experimental import pallas as pl
from jax.experimental.pallas import tpu as pltpu
from jax.experimental.pallas import tpu_sc as plsc

N = 100000
D1 = 512
D2 = 768
M = 1000
K = 10
RB = 7168
GA = 14
NPAD = GA * RB
NW = 16
CHUNK = NPAD // NW
CPW = 1024 // NW
EQ_CAP = 1008
NEG_INF = float("-inf")
IMIN = -2**31


def _sortable_key(s):
    bits = lax.bitcast_convert_type(s, jnp.int32)
    m = lax.shift_right_arithmetic(bits, 31)
    return bits ^ (m & jnp.int32(0x7FFFFFFF))


def _stage1_body(x_ref, t1_ref, keys_out, meta_out, skeys):
    b = pl.program_id(0)
    x = x_ref[...]
    t1 = t1_ref[...]
    dot = jnp.sum(x * t1[None, :], axis=1)
    na = jnp.sqrt(jnp.sum(x * x, axis=1))
    nb = jnp.sqrt(jnp.sum(t1 * t1))
    s = dot / jnp.maximum(na * nb, jnp.float32(1e-8))
    key = _sortable_key(s)
    rows = b * RB + lax.broadcasted_iota(jnp.int32, (RB,), 0)
    key = jnp.where(rows < N, key, jnp.int32(IMIN))
    keys_out[...] = key
    skeys[pl.ds(56 * b, 56), :] = key.reshape(56, 128)

    @pl.when(b == GA - 1)
    def _():
        allk = skeys[...]

        def cnt_ge(t):
            return jnp.sum((allk >= t).astype(jnp.int32))

        def bs_step(_, c):
            lo, hi = c
            mid = (lo >> 1) + (hi >> 1) + (lo & hi & 1)
            ge = cnt_ge(mid) >= M
            return jnp.where(ge, mid, lo), jnp.where(ge, hi, mid)

        lo, hi = lax.fori_loop(0, 32, bs_step, (jnp.int32(IMIN), jnp.int32(2**31 - 1)))
        t_key = lo
        c_gt = cnt_ge(t_key + 1)
        t_need = M - c_gt
        i = lax.broadcasted_iota(jnp.int32, (128,), 0)
        meta = jnp.where(i == 0, t_key, jnp.where(i == 1, c_gt, jnp.where(i == 2, t_need, 0)))
        meta_out[...] = meta


_stage1 = pl.pallas_call(
    _stage1_body,
    grid=(GA,),
    in_specs=[
        pl.BlockSpec((RB, D1), lambda b: (b, 0)),
        pl.BlockSpec((D1,), lambda b: (0,)),
    ],
    out_specs=[
        pl.BlockSpec((RB,), lambda b: (b,)),
        pl.BlockSpec((128,), lambda b: (0,)),
    ],
    out_shape=[
        jax.ShapeDtypeStruct((NPAD,), jnp.int32),
        jax.ShapeDtypeStruct((128,), jnp.int32),
    ],
    scratch_shapes=[pltpu.VMEM((GA * 56, 128), jnp.int32)],
)


def _lane(vec, i):
    return jnp.sum(jnp.where(lax.iota(jnp.int32, 16) == i, vec, 0))


def _sc_body(keys_hbm, meta_hbm, stage2_hbm, cand_hbm, emb_hbm,
             keys_v, gt_v, eq_v, idx_v, meta_v, cnts_v, ids_v, rows_v,
             shared_cnts, shared_cand, sem):
    wid = lax.axis_index("s")
    lane16 = lax.iota(jnp.int32, 16)

    pltpu.sync_copy(keys_hbm.at[pl.ds(wid * CHUNK, CHUNK)], keys_v)
    pltpu.sync_copy(meta_hbm.at[pl.ds(0, 16)], meta_v)
    mv = meta_v[...]
    t_key = _lane(mv, 0)
    c_gt_tot = _lane(mv, 1)
    t_need = _lane(mv, 2)
    t_vec = jnp.full((16,), t_key, dtype=jnp.int32)

    def comp_step(i, carry):
        cg_vec, ce_vec = carry
        v = keys_v[pl.ds(i * 16, 16)]
        gids = wid * CHUNK + i * 16 + lane16
        m_gt = v > t_vec
        m_eq = v == t_vec
        pos_g = cg_vec + plsc.cumsum(m_gt.astype(jnp.int32)) - 1
        plsc.store_scatter(gt_v, [pos_g], gids, mask=m_gt)
        pos_e = ce_vec + plsc.cumsum(m_eq.astype(jnp.int32)) - 1
        pos_e = jnp.minimum(pos_e, EQ_CAP + lane16)
        plsc.store_scatter(eq_v, [pos_e], gids, mask=m_eq)
        cg_vec = cg_vec + plsc.all_reduce_population_count(m_gt)
        ce_vec = ce_vec + plsc.all_reduce_population_count(m_eq)
        return cg_vec, ce_vec

    zero16 = jnp.zeros((16,), jnp.int32)
    cg_vec, ce_vec = lax.fori_loop(0, CHUNK // 16, comp_step, (zero16, zero16))
    ce_vec = jnp.minimum(ce_vec, EQ_CAP)

    cnts_v[pl.ds(0, 16)] = cg_vec
    cnts_v[pl.ds(16, 16)] = ce_vec
    pltpu.sync_copy(cnts_v.at[pl.ds(0, 16)], shared_cnts.at[pl.ds(wid * 16, 16)])
    pltpu.sync_copy(cnts_v.at[pl.ds(16, 16)], shared_cnts.at[pl.ds((NW + wid) * 16, 16)])
    plsc.subcore_barrier()

    pltpu.sync_copy(shared_cnts, cnts_v)

    def pfx_step(w, carry):
        og, oe = carry
        cg = jnp.max(cnts_v[pl.ds(w * 16, 16)])
        ce = jnp.max(cnts_v[pl.ds((NW + w) * 16, 16)])
        take = (w < wid).astype(jnp.int32)
        return og + take * cg, oe + take * ce

    og, oe = lax.fori_loop(0, NW, pfx_step, (jnp.int32(0), jnp.int32(0)))
    cg_w = jnp.max(cg_vec)
    ce_w = jnp.max(ce_vec)
    acc_e = jnp.clip(t_need - oe, 0, ce_w)

    def scatter_list(list_ref, base, count, tag):
        def fill(p, carry):
            j = p * 16 + lane16
            dump = 1024 + ((wid * (37 + tag) + j) & 1023)
            idx_v[pl.ds(p * 16, 16)] = jnp.clip(
                jnp.where(j < count, base + j, dump), 0, 2047)
            return carry
        lax.fori_loop(0, 1024 // 16, fill, jnp.int32(0))
        pltpu.async_copy(list_ref, shared_cand.at[idx_v], sem).wait()

    scatter_list(gt_v, og, cg_w, 0)
    scatter_list(eq_v, c_gt_tot + oe, acc_e, 7)
    plsc.subcore_barrier()

    pltpu.sync_copy(shared_cand.at[pl.ds(wid * CPW, CPW)], ids_v)
    for q in range(CPW // 16):
        ids_v[pl.ds(q * 16, 16)] = jnp.clip(ids_v[pl.ds(q * 16, 16)], 0, N - 1)
    pltpu.sync_copy(ids_v, cand_hbm.at[pl.ds(wid * CPW, CPW)])
    pltpu.async_copy(stage2_hbm.at[ids_v], rows_v, sem).wait()
    pltpu.sync_copy(rows_v, emb_hbm.at[pl.ds(wid * CPW, CPW)])


@functools.cache
def _sc_stage():
    return pl.kernel(
        _sc_body,
        out_type=(jax.ShapeDtypeStruct((1024,), jnp.int32),
                  jax.ShapeDtypeStruct((1024, D2), jnp.float32)),
        mesh=plsc.VectorSubcoreMesh(core_axis_name="c", subcore_axis_name="s",
                                    num_cores=1, num_subcores=16),
        compiler_params=pltpu.CompilerParams(needs_layout_passes=False),
        scratch_types=[
            pltpu.VMEM((CHUNK,), jnp.int32),
            pltpu.VMEM((1024,), jnp.int32),
            pltpu.VMEM((1024,), jnp.int32),
            pltpu.VMEM((1024,), jnp.int32),
            pltpu.VMEM((16,), jnp.int32),
            pltpu.VMEM((2 * NW * 16,), jnp.int32),
            pltpu.VMEM((CPW,), jnp.int32),
            pltpu.VMEM((CPW, D2), jnp.float32),
            pltpu.VMEM_SHARED((2 * NW * 16,), jnp.int32),
            pltpu.VMEM_SHARED((2048,), jnp.int32),
            pltpu.SemaphoreType.DMA,
        ],
    )


def _stage2_body(emb_ref, t2_ref, cand_ref, out_ref):
    x = emb_ref[...]
    t2 = t2_ref[...]
    dot = jnp.sum(x * t2[None, :], axis=1)
    na = jnp.sqrt(jnp.sum(x * x, axis=1))
    nb = jnp.sqrt(jnp.sum(t2 * t2))
    s = (dot / jnp.maximum(na * nb, jnp.float32(1e-8))).reshape(8, 128)
    flat = (lax.broadcasted_iota(jnp.int32, (8, 128), 0) * 128
            + lax.broadcasted_iota(jnp.int32, (8, 128), 1))
    s = jnp.where(flat < M, s, jnp.float32(NEG_INF))
    cand = cand_ref[...].reshape(8, 128)
    acc = jnp.zeros((8, 128), jnp.int32)
    for k in range(K):
        mval = jnp.max(s)
        p = jnp.min(jnp.where(s == mval, flat, jnp.int32(4096)))
        gid = jnp.sum(jnp.where(flat == p, cand, 0))
        acc = jnp.where(flat == k, gid, acc)
        s = jnp.where(flat == p, jnp.float32(NEG_INF), s)
    out_ref[...] = acc


_stage2 = pl.pallas_call(
    _stage2_body,
    in_specs=[
        pl.BlockSpec((1024, D2), lambda: (0, 0)),
        pl.BlockSpec((D2,), lambda: (0,)),
        pl.BlockSpec((1024,), lambda: (0,)),
    ],
    out_specs=pl.BlockSpec((8, 128), lambda: (0, 0)),
    out_shape=jax.ShapeDtypeStruct((8, 128), jnp.int32),
)


def kernel(base_images_emb, stage2_images_emb, text_emb_stage1, text_emb_stage2, topm, topk):
    keys, meta = _stage1(base_images_emb, text_emb_stage1)
    cand, cand_emb = _sc_stage()(keys, meta, stage2_images_emb)
    out = _stage2(cand_emb, text_emb_stage2, cand)
    return out[0, :K]
